# baseline (speedup 1.0000x reference)
import functools

import jax
import jax.numpy as jnp
from jax import lax
from jax.experimental import pallas as pl
from jax.experimental.pallas import tpu as pltpu
from jax.experimental.pallas import tpu_sc as plsc

MAX_LEN = 8192
HALF = MAX_LEN // 2
D_MODEL = 768
BATCH = 4
NUM_CORES = 2
NUM_SUBCORES = 16
NW = NUM_CORES * NUM_SUBCORES
ROWS_PER_W = HALF // NW
NCHUNK = 4
CHUNK = ROWS_PER_W // NCHUNK

_mesh = plsc.VectorSubcoreMesh(core_axis_name="c", subcore_axis_name="s")


@functools.partial(
    pl.kernel,
    mesh=_mesh,
    out_type=jax.ShapeDtypeStruct((BATCH * MAX_LEN, D_MODEL), jnp.float32),
    scratch_types=(
        [pltpu.VMEM((CHUNK, D_MODEL), jnp.float32)] * NCHUNK
        + [pltpu.VMEM((CHUNK,), jnp.int32)] * (NCHUNK * BATCH)
        + [
            pltpu.VMEM((16, D_MODEL), jnp.float32),
            pltpu.VMEM((16,), jnp.int32),
            pltpu.VMEM((16,), jnp.int32),
            pltpu.SemaphoreType.DMA,
            pltpu.SemaphoreType.DMA,
            pltpu.SemaphoreType.DMA,
        ]
    ),
)
def _rel_pos_emb(table_hbm, out_hbm, *refs):
    rows = refs[:NCHUNK]
    idx_flat = refs[NCHUNK:NCHUNK + NCHUNK * BATCH]
    ridx = [idx_flat[c * BATCH:(c + 1) * BATCH] for c in range(NCHUNK)]
    spec_v, gidx, oidx, sem, psem, rsem = refs[NCHUNK + NCHUNK * BATCH:]

    wid = lax.axis_index("s") * NUM_CORES + lax.axis_index("c")
    rbase = wid * ROWS_PER_W

    def read_desc(c):
        return pltpu.make_async_copy(
            table_hbm.at[pl.ds(rbase + c * CHUNK, CHUNK)], rows[c], rsem)

    for c in range(NCHUNK):
        read_desc(c).start()

    for c in range(NCHUNK):
        for b in range(BATCH):
            for t in range(CHUNK // 16):
                head = b * MAX_LEN + HALF - rbase - c * CHUNK - t * 16
                ridx[c][b][pl.ds(t * 16, 16)] = head - lax.iota(jnp.int32, 16)

    copies = []
    for c in range(NCHUNK):
        read_desc(c).wait()
        for b in range(BATCH):
            copies.append(pltpu.async_copy(rows[c], out_hbm.at[ridx[c][b]],
                                           sem))
            copies.append(pltpu.async_copy(
                rows[c],
                out_hbm.at[pl.ds(b * MAX_LEN + HALF + rbase + c * CHUNK,
                                 CHUNK)],
                sem))

    @pl.when(wid < BATCH)
    def _patch():
        gidx[...] = HALF - lax.iota(jnp.int32, 16)
        oidx[...] = wid * MAX_LEN + lax.iota(jnp.int32, 16)
        pltpu.async_copy(table_hbm.at[gidx], spec_v, psem).wait()
        pltpu.async_copy(spec_v, out_hbm.at[oidx], psem).wait()

    for c in copies:
        c.wait()


def kernel(x, table):
    del x
    out = _rel_pos_emb(table)
    return out.reshape(BATCH, MAX_LEN, D_MODEL)


# speedup vs baseline: 1.0122x; 1.0122x over previous
import functools

import jax
import jax.numpy as jnp
from jax import lax
from jax.experimental import pallas as pl
from jax.experimental.pallas import tpu as pltpu
from jax.experimental.pallas import tpu_sc as plsc

MAX_LEN = 8192
HALF = MAX_LEN // 2
D_MODEL = 768
BATCH = 4
NUM_CORES = 2
NUM_SUBCORES = 16
NW = NUM_CORES * NUM_SUBCORES
ROWS_PER_W = HALF // NW
CHUNK = ROWS_PER_W // 2

_mesh = plsc.VectorSubcoreMesh(core_axis_name="c", subcore_axis_name="s")


@functools.partial(
    pl.kernel,
    mesh=_mesh,
    out_type=jax.ShapeDtypeStruct((BATCH * MAX_LEN, D_MODEL), jnp.float32),
    scratch_types=[
        pltpu.VMEM((CHUNK, D_MODEL), jnp.float32),
        pltpu.VMEM((CHUNK, D_MODEL), jnp.float32),
        pltpu.VMEM((CHUNK,), jnp.int32),
        pltpu.VMEM((CHUNK,), jnp.int32),
        pltpu.VMEM((CHUNK,), jnp.int32),
        pltpu.VMEM((CHUNK,), jnp.int32),
        pltpu.VMEM((CHUNK,), jnp.int32),
        pltpu.VMEM((CHUNK,), jnp.int32),
        pltpu.VMEM((CHUNK,), jnp.int32),
        pltpu.VMEM((CHUNK,), jnp.int32),
        pltpu.VMEM((16, D_MODEL), jnp.float32),
        pltpu.VMEM((16,), jnp.int32),
        pltpu.VMEM((16,), jnp.int32),
        pltpu.SemaphoreType.DMA,
        pltpu.SemaphoreType.DMA,
        pltpu.SemaphoreType.DMA,
    ],
)
def _rel_pos_emb(table_hbm, out_hbm, rows_a, rows_b,
                 ia0, ia1, ia2, ia3, ib0, ib1, ib2, ib3,
                 spec_v, gidx, oidx, sem, psem, rsem):
    wid = lax.axis_index("s") * NUM_CORES + lax.axis_index("c")
    rbase = wid * ROWS_PER_W

    rows = [rows_a, rows_b]
    ridx = [[ia0, ia1, ia2, ia3], [ib0, ib1, ib2, ib3]]

    def read_desc(c):
        return pltpu.make_async_copy(
            table_hbm.at[pl.ds(rbase + c * CHUNK, CHUNK)], rows[c], rsem)

    read_desc(0).start()
    read_desc(1).start()

    for c in range(2):
        for b in range(BATCH):
            for t in range(CHUNK // 16):
                head = b * MAX_LEN + HALF - rbase - c * CHUNK - t * 16
                ridx[c][b][pl.ds(t * 16, 16)] = head - lax.iota(jnp.int32, 16)

    copies = []
    for c in range(2):
        read_desc(c).wait()
        for b in range(BATCH):
            copies.append(pltpu.async_copy(rows[c], out_hbm.at[ridx[c][b]],
                                           sem))
            copies.append(pltpu.async_copy(
                rows[c],
                out_hbm.at[pl.ds(b * MAX_LEN + HALF + rbase + c * CHUNK,
                                 CHUNK)],
                sem))

    @pl.when(wid < BATCH)
    def _patch():
        gidx[...] = HALF - lax.iota(jnp.int32, 16)
        oidx[...] = wid * MAX_LEN + lax.iota(jnp.int32, 16)
        pltpu.async_copy(table_hbm.at[gidx], spec_v, psem).wait()
        pltpu.async_copy(spec_v, out_hbm.at[oidx], psem).wait()

    for c in copies:
        c.wait()


def kernel(x, table):
    del x
    out = _rel_pos_emb(table)
    return out.reshape(BATCH, MAX_LEN, D_MODEL)
